# Initial kernel scaffold; baseline (speedup 1.0000x reference)
import jax
import jax.numpy as jnp
from jax import lax
from jax.experimental import pallas as pl
from jax.experimental.pallas import tpu as pltpu
from jax.experimental.pallas import tpu_sc as plsc

N_NODES = 10000
N_EDGES = 320000
D_NODE = 128
D_EDGE = 16

NC = 2
NS = 16
NW = NC * NS
EW = N_EDGES // NW
CW = 125
KC = EW // CW
RS = 5
SL = KC // RS
ROWS_T = N_NODES // NS
CNT_CH = 2000


def _sc_body(row_hbm, col_hbm, attr_hbm, z16_hbm, z1_hbm,
             rec_out, sent_out, cntr_out, cnts_out,
             idx_row, idx_col, attr_buf, ones_v,
             acc_rec, acc_sent, cnt_rec, cnt_sent):
    cid = lax.axis_index("c")
    sid = lax.axis_index("s")
    wid = cid * NS + sid

    pltpu.sync_copy(row_hbm.at[wid], idx_row)
    pltpu.sync_copy(col_hbm.at[wid], idx_col)

    for i in range(8):
        ones_v[pl.ds(i * 16, 16)] = jnp.full((16,), 1.0, jnp.float32)

    r0 = sid * ROWS_T
    pltpu.sync_copy(z16_hbm.at[pl.ds(r0, ROWS_T)], acc_rec.at[pl.ds(r0, ROWS_T)])
    pltpu.sync_copy(z16_hbm.at[pl.ds(r0, ROWS_T)], acc_sent.at[pl.ds(r0, ROWS_T)])
    for k in range(5):
        @pl.when(sid == k)
        def _():
            pltpu.sync_copy(z1_hbm.at[pl.ds(CNT_CH * k, CNT_CH)],
                            cnt_rec.at[pl.ds(CNT_CH * k, CNT_CH)])

        @pl.when(sid == 5 + k)
        def _():
            pltpu.sync_copy(z1_hbm.at[pl.ds(CNT_CH * k, CNT_CH)],
                            cnt_sent.at[pl.ds(CNT_CH * k, CNT_CH)])
    plsc.subcore_barrier()

    def slab_step(s, carry):
        pltpu.sync_copy(attr_hbm.at[wid, s], attr_buf)
        for j in range(RS):
            jj = s * RS + j
            pltpu.sync_copy(attr_buf.at[j], acc_rec.at[idx_row.at[jj]], add=True)
            pltpu.sync_copy(attr_buf.at[j], acc_sent.at[idx_col.at[jj]], add=True)
            pltpu.sync_copy(ones_v.at[pl.ds(0, CW)], cnt_rec.at[idx_row.at[jj]], add=True)
            pltpu.sync_copy(ones_v.at[pl.ds(0, CW)], cnt_sent.at[idx_col.at[jj]], add=True)
        return carry

    lax.fori_loop(0, SL, slab_step, 0)
    plsc.subcore_barrier()

    pltpu.sync_copy(acc_rec.at[pl.ds(r0, ROWS_T)], rec_out.at[cid, pl.ds(r0, ROWS_T)])
    pltpu.sync_copy(acc_sent.at[pl.ds(r0, ROWS_T)], sent_out.at[cid, pl.ds(r0, ROWS_T)])
    for k in range(5):
        @pl.when(sid == k)
        def _():
            pltpu.sync_copy(cnt_rec.at[pl.ds(CNT_CH * k, CNT_CH)],
                            cntr_out.at[cid, pl.ds(CNT_CH * k, CNT_CH)])

        @pl.when(sid == 5 + k)
        def _():
            pltpu.sync_copy(cnt_sent.at[pl.ds(CNT_CH * k, CNT_CH)],
                            cnts_out.at[cid, pl.ds(CNT_CH * k, CNT_CH)])


def _make_sc_scatter():
    return pl.kernel(
        _sc_body,
        out_type=(
            jax.ShapeDtypeStruct((NC, N_NODES, D_EDGE), jnp.float32),
            jax.ShapeDtypeStruct((NC, N_NODES, D_EDGE), jnp.float32),
            jax.ShapeDtypeStruct((NC, N_NODES), jnp.float32),
            jax.ShapeDtypeStruct((NC, N_NODES), jnp.float32),
        ),
        mesh=plsc.VectorSubcoreMesh(core_axis_name="c", subcore_axis_name="s"),
        scratch_types=[
            pltpu.VMEM((KC, CW), jnp.int32),
            pltpu.VMEM((KC, CW), jnp.int32),
            pltpu.VMEM((RS, CW, D_EDGE), jnp.float32),
            pltpu.VMEM((128,), jnp.float32),
            pltpu.VMEM_SHARED((N_NODES, D_EDGE), jnp.float32),
            pltpu.VMEM_SHARED((N_NODES, D_EDGE), jnp.float32),
            pltpu.VMEM_SHARED((N_NODES,), jnp.float32),
            pltpu.VMEM_SHARED((N_NODES,), jnp.float32),
        ],
    )


def _tc_body(x_ref, rec_ref, sent_ref, cr_ref, cs_ref,
             w1x_ref, w1r_ref, w1s_ref, b1_ref, w2_ref, b2_ref, o_ref):
    rec = rec_ref[0] + rec_ref[1]
    sent = sent_ref[0] + sent_ref[1]
    cr = jnp.maximum(cr_ref[0] + cr_ref[1], 1.0)
    cs = jnp.maximum(cs_ref[0] + cs_ref[1], 1.0)
    rec = rec / cr
    sent = sent / cs
    h = jnp.dot(x_ref[...], w1x_ref[...], preferred_element_type=jnp.float32)
    h = h + jnp.dot(rec, w1r_ref[...], preferred_element_type=jnp.float32)
    h = h + jnp.dot(sent, w1s_ref[...], preferred_element_type=jnp.float32)
    h = h + b1_ref[...]
    h = jnp.where(h >= 0, h, 0.01 * h)
    o_ref[...] = jnp.dot(h, w2_ref[...], preferred_element_type=jnp.float32) + b2_ref[...]


def kernel(x, edge_index, edge_attr, W1, b1, W2, b2):
    row = edge_index[0].astype(jnp.int32).reshape(NW, KC, CW)
    col = edge_index[1].astype(jnp.int32).reshape(NW, KC, CW)
    attr = edge_attr.reshape(NW, SL, RS, CW, D_EDGE)
    z16 = jnp.zeros((N_NODES, D_EDGE), jnp.float32)
    z1 = jnp.zeros((N_NODES,), jnp.float32)

    rec_p, sent_p, cntr_p, cnts_p = _make_sc_scatter()(row, col, attr, z16, z1)

    out = pl.pallas_call(
        _tc_body,
        out_shape=jax.ShapeDtypeStruct((N_NODES, D_NODE), jnp.float32),
    )(x, rec_p, sent_p,
      cntr_p.reshape(NC, N_NODES, 1), cnts_p.reshape(NC, N_NODES, 1),
      W1[:D_NODE], W1[D_NODE:D_NODE + D_EDGE], W1[D_NODE + D_EDGE:],
      b1.reshape(1, D_NODE), W2, b2.reshape(1, D_NODE))
    return out


# speedup vs baseline: 7.8392x; 7.8392x over previous
import jax
import jax.numpy as jnp
from jax import lax
from jax.experimental import pallas as pl
from jax.experimental.pallas import tpu as pltpu
from jax.experimental.pallas import tpu_sc as plsc

N_NODES = 10000
N_EDGES = 320000
D_NODE = 128
D_EDGE = 16

NP = 10240
NC = 2
NS = 16
NW = NC * NS
EW = N_EDGES // NW
CW = 125
KC = EW // CW
RS = 5
SL = KC // RS
ROWS_T = NP // NS


def _sc_body(row_hbm, col_hbm, attr_hbm, z16_hbm, z8_hbm, on8_hbm,
             rec_out, sent_out, cnt_out,
             idx_row, idx_col, attr_buf, ones_v,
             acc_rec, acc_sent, acc_cnt):
    cid = lax.axis_index("c")
    sid = lax.axis_index("s")
    wid = cid * NS + sid

    pltpu.sync_copy(row_hbm.at[wid], idx_row)
    pltpu.sync_copy(col_hbm.at[wid], idx_col)

    pltpu.sync_copy(on8_hbm, ones_v)

    r0 = sid * ROWS_T
    for acc in (acc_rec, acc_sent):
        pltpu.sync_copy(z16_hbm.at[pl.ds(r0, ROWS_T)], acc.at[pl.ds(r0, ROWS_T)])
    pltpu.sync_copy(z8_hbm.at[pl.ds(r0, ROWS_T)], acc_cnt.at[pl.ds(r0, ROWS_T)])
    plsc.subcore_barrier()

    def slab_step(s, carry):
        pltpu.sync_copy(attr_hbm.at[wid, s], attr_buf)
        for j in range(RS):
            jj = s * RS + j
            pltpu.sync_copy(attr_buf.at[j], acc_rec.at[idx_row.at[jj]], add=True)
            pltpu.sync_copy(attr_buf.at[j], acc_sent.at[idx_col.at[jj]], add=True)
            pltpu.sync_copy(ones_v.at[0], acc_cnt.at[idx_row.at[jj]], add=True)
            pltpu.sync_copy(ones_v.at[1], acc_cnt.at[idx_col.at[jj]], add=True)
        return carry

    lax.fori_loop(0, SL, slab_step, 0)
    plsc.subcore_barrier()

    pltpu.sync_copy(acc_rec.at[pl.ds(r0, ROWS_T)], rec_out.at[cid, pl.ds(r0, ROWS_T)])
    pltpu.sync_copy(acc_sent.at[pl.ds(r0, ROWS_T)], sent_out.at[cid, pl.ds(r0, ROWS_T)])
    pltpu.sync_copy(acc_cnt.at[pl.ds(r0, ROWS_T)], cnt_out.at[cid, pl.ds(r0, ROWS_T)])


def _make_sc_scatter():
    part = jax.ShapeDtypeStruct((NC, NP, D_EDGE), jnp.float32)
    cpart = jax.ShapeDtypeStruct((NC, NP, 8), jnp.float32)
    return pl.kernel(
        _sc_body,
        out_type=(part, part, cpart),
        mesh=plsc.VectorSubcoreMesh(core_axis_name="c", subcore_axis_name="s",
                                    num_cores=NC, num_subcores=NS),
        compiler_params=pltpu.CompilerParams(use_tc_tiling_on_sc=False),
        scratch_types=[
            pltpu.VMEM((KC, CW), jnp.int32),
            pltpu.VMEM((KC, CW), jnp.int32),
            pltpu.VMEM((RS, CW, D_EDGE), jnp.float32),
            pltpu.VMEM((2, CW, 8), jnp.float32),
            pltpu.VMEM_SHARED((NP, D_EDGE), jnp.float32),
            pltpu.VMEM_SHARED((NP, D_EDGE), jnp.float32),
            pltpu.VMEM_SHARED((NP, 8), jnp.float32),
        ],
    )


def _tc_body(x_ref, rec_ref, sent_ref, cnt_ref,
             w1x_ref, w1r_ref, w1s_ref, b1_ref, w2_ref, b2_ref, o_ref):
    rec = (rec_ref[0] + rec_ref[1])[:N_NODES]
    sent = (sent_ref[0] + sent_ref[1])[:N_NODES]
    cnt = cnt_ref[0] + cnt_ref[1]
    cr = jnp.maximum(cnt[:N_NODES, 0:1], 1.0)
    cs = jnp.maximum(cnt[:N_NODES, 4:5], 1.0)
    rec = rec / cr
    sent = sent / cs
    h = jnp.dot(x_ref[...], w1x_ref[...], preferred_element_type=jnp.float32)
    h = h + jnp.dot(rec, w1r_ref[...], preferred_element_type=jnp.float32)
    h = h + jnp.dot(sent, w1s_ref[...], preferred_element_type=jnp.float32)
    h = h + b1_ref[...]
    h = jnp.where(h >= 0, h, 0.01 * h)
    o_ref[...] = jnp.dot(h, w2_ref[...], preferred_element_type=jnp.float32) + b2_ref[...]


def kernel(x, edge_index, edge_attr, W1, b1, W2, b2):
    row = edge_index[0].astype(jnp.int32).reshape(NW, KC, CW)
    col = edge_index[1].astype(jnp.int32).reshape(NW, KC, CW)
    attr = edge_attr.reshape(NW, SL, RS, CW, D_EDGE)
    z16 = jnp.zeros((NP, D_EDGE), jnp.float32)
    z8 = jnp.zeros((NP, 8), jnp.float32)
    lane = jnp.arange(8)
    on8 = jnp.stack([jnp.where(lane < 4, 1.0, 0.0),
                     jnp.where(lane >= 4, 1.0, 0.0)]).astype(jnp.float32)
    on8 = jnp.broadcast_to(on8[:, None, :], (2, CW, 8))

    rec_p, sent_p, cnt_p = _make_sc_scatter()(row, col, attr, z16, z8, on8)

    out = pl.pallas_call(
        _tc_body,
        out_shape=jax.ShapeDtypeStruct((N_NODES, D_NODE), jnp.float32),
    )(x, rec_p, sent_p, cnt_p,
      W1[:D_NODE], W1[D_NODE:D_NODE + D_EDGE], W1[D_NODE + D_EDGE:],
      b1.reshape(1, D_NODE), W2, b2.reshape(1, D_NODE))
    return out


# speedup vs baseline: 10.5596x; 1.3470x over previous
import jax
import jax.numpy as jnp
from jax import lax
from jax.experimental import pallas as pl
from jax.experimental.pallas import tpu as pltpu
from jax.experimental.pallas import tpu_sc as plsc

N_NODES = 10000
N_EDGES = 320000
D_NODE = 128
D_EDGE = 16

NP = 10240
NC = 2
NS = 16
NW = NC * NS
EW = N_EDGES // NW
CW = 125
KC = EW // CW
SB = 1000
XW = SB + 9
RS = SB // CW
SL = EW // SB
ROWS_T = NP // NS


def _sc_body(row_hbm, col_hbm, attr_hbm, z16_hbm, z8_hbm, on8_hbm,
             rec_out, sent_out, cnt_out,
             idx_row, idx_col, xbuf, tbuf, ones_v,
             acc_rec, acc_sent, acc_cnt):
    cid = lax.axis_index("c")
    sid = lax.axis_index("s")
    wid = cid * NS + sid

    pltpu.sync_copy(row_hbm.at[wid], idx_row)
    pltpu.sync_copy(col_hbm.at[wid], idx_col)

    pltpu.sync_copy(on8_hbm, ones_v)

    r0 = sid * ROWS_T
    for acc in (acc_rec, acc_sent):
        pltpu.sync_copy(z16_hbm.at[pl.ds(r0, ROWS_T)], acc.at[pl.ds(r0, ROWS_T)])
    pltpu.sync_copy(z8_hbm.at[pl.ds(r0, ROWS_T)], acc_cnt.at[pl.ds(r0, ROWS_T)])
    plsc.subcore_barrier()

    lane = lax.iota(jnp.int32, 16)

    def slab_step(s, carry):
        e0 = wid * EW + s * SB
        pltpu.sync_copy(attr_hbm.at[:, pl.ds(e0, SB)], xbuf.at[:, pl.ds(0, SB)])

        def xpose(e, carry):
            vec = plsc.load_gather(xbuf, [lane, jnp.full((16,), e, jnp.int32)])
            plsc.store_scatter(tbuf, [jnp.full((16,), e, jnp.int32), lane], vec)
            return carry

        lax.fori_loop(0, SB, xpose, 0)

        for j in range(RS):
            jj = s * RS + j
            src = tbuf.at[pl.ds(j * CW, CW)]
            pltpu.sync_copy(src, acc_rec.at[idx_row.at[jj]], add=True)
            pltpu.sync_copy(src, acc_sent.at[idx_col.at[jj]], add=True)
            pltpu.sync_copy(ones_v.at[0], acc_cnt.at[idx_row.at[jj]], add=True)
            pltpu.sync_copy(ones_v.at[1], acc_cnt.at[idx_col.at[jj]], add=True)
        return carry

    lax.fori_loop(0, SL, slab_step, 0)
    plsc.subcore_barrier()

    pltpu.sync_copy(acc_rec.at[pl.ds(r0, ROWS_T)], rec_out.at[cid, pl.ds(r0, ROWS_T)])
    pltpu.sync_copy(acc_sent.at[pl.ds(r0, ROWS_T)], sent_out.at[cid, pl.ds(r0, ROWS_T)])
    pltpu.sync_copy(acc_cnt.at[pl.ds(r0, ROWS_T)], cnt_out.at[cid, pl.ds(r0, ROWS_T)])


def _make_sc_scatter():
    part = jax.ShapeDtypeStruct((NC, NP, D_EDGE), jnp.float32)
    cpart = jax.ShapeDtypeStruct((NC, NP, 8), jnp.float32)
    return pl.kernel(
        _sc_body,
        out_type=(part, part, cpart),
        mesh=plsc.VectorSubcoreMesh(core_axis_name="c", subcore_axis_name="s",
                                    num_cores=NC, num_subcores=NS),
        compiler_params=pltpu.CompilerParams(use_tc_tiling_on_sc=False,
                                             needs_layout_passes=False),
        scratch_types=[
            pltpu.VMEM((KC, CW), jnp.int32),
            pltpu.VMEM((KC, CW), jnp.int32),
            pltpu.VMEM((D_EDGE, XW), jnp.float32),
            pltpu.VMEM((SB, D_EDGE), jnp.float32),
            pltpu.VMEM((2, CW, 8), jnp.float32),
            pltpu.VMEM_SHARED((NP, D_EDGE), jnp.float32),
            pltpu.VMEM_SHARED((NP, D_EDGE), jnp.float32),
            pltpu.VMEM_SHARED((NP, 8), jnp.float32),
        ],
    )


def _tc_body(x_ref, rec_ref, sent_ref, cnt_ref,
             w1x_ref, w1r_ref, w1s_ref, b1_ref, w2_ref, b2_ref, o_ref):
    rec = (rec_ref[0] + rec_ref[1])[:N_NODES]
    sent = (sent_ref[0] + sent_ref[1])[:N_NODES]
    cnt = cnt_ref[0] + cnt_ref[1]
    cr = jnp.maximum(cnt[:N_NODES, 0:1], 1.0)
    cs = jnp.maximum(cnt[:N_NODES, 4:5], 1.0)
    rec = rec / cr
    sent = sent / cs
    h = jnp.dot(x_ref[...], w1x_ref[...], preferred_element_type=jnp.float32)
    h = h + jnp.dot(rec, w1r_ref[...], preferred_element_type=jnp.float32)
    h = h + jnp.dot(sent, w1s_ref[...], preferred_element_type=jnp.float32)
    h = h + b1_ref[...]
    h = jnp.where(h >= 0, h, 0.01 * h)
    o_ref[...] = jnp.dot(h, w2_ref[...], preferred_element_type=jnp.float32) + b2_ref[...]


def kernel(x, edge_index, edge_attr, W1, b1, W2, b2):
    row = edge_index[0].astype(jnp.int32).reshape(NW, KC, CW)
    col = edge_index[1].astype(jnp.int32).reshape(NW, KC, CW)
    z16 = jnp.zeros((NP, D_EDGE), jnp.float32)
    z8 = jnp.zeros((NP, 8), jnp.float32)
    lane = jnp.arange(8)
    on8 = jnp.stack([jnp.where(lane < 4, 1.0, 0.0),
                     jnp.where(lane >= 4, 1.0, 0.0)]).astype(jnp.float32)
    on8 = jnp.broadcast_to(on8[:, None, :], (2, CW, 8))

    rec_p, sent_p, cnt_p = _make_sc_scatter()(row, col, edge_attr.T, z16, z8, on8)

    out = pl.pallas_call(
        _tc_body,
        out_shape=jax.ShapeDtypeStruct((N_NODES, D_NODE), jnp.float32),
    )(x, rec_p, sent_p, cnt_p,
      W1[:D_NODE], W1[D_NODE:D_NODE + D_EDGE], W1[D_NODE + D_EDGE:],
      b1.reshape(1, D_NODE), W2, b2.reshape(1, D_NODE))
    return out


# speedup vs baseline: 12.7019x; 1.2029x over previous
import jax
import jax.numpy as jnp
from jax import lax
from jax.experimental import pallas as pl
from jax.experimental.pallas import tpu as pltpu
from jax.experimental.pallas import tpu_sc as plsc

N_NODES = 10000
N_EDGES = 320000
D_NODE = 128
D_EDGE = 16

NP = 10240
NC = 2
NS = 16
NW = NC * NS
EW = N_EDGES // NW
CW = 125
KC = EW // CW
SB = 1000
XW = SB + 9
RS = SB // CW
SL = EW // SB
ROWS_T = NP // NS


def _sc_body(row_hbm, col_hbm, attr_hbm, z16_hbm, z8_hbm, on8_hbm,
             rec_out, sent_out, cnt_out,
             idx_row, idx_col, xbuf, tbuf, ones_v,
             acc_rec, acc_sent, acc_cnt, dma_sem, str_sem):
    cid = lax.axis_index("c")
    sid = lax.axis_index("s")
    wid = cid * NS + sid


    pltpu.sync_copy(on8_hbm, ones_v)

    r0 = sid * ROWS_T
    for acc in (acc_rec, acc_sent):
        pltpu.sync_copy(z16_hbm.at[pl.ds(r0, ROWS_T)], acc.at[pl.ds(r0, ROWS_T)])
    pltpu.sync_copy(z8_hbm.at[pl.ds(r0, ROWS_T)], acc_cnt.at[pl.ds(r0, ROWS_T)])
    plsc.subcore_barrier()

    lane = lax.iota(jnp.int32, 16)

    def _slab_refs(s):
        e0 = wid * EW + s * SB
        return attr_hbm.at[:, pl.ds(e0, SB)], xbuf.at[:, pl.ds(0, SB)]

    def _idx_refs(s, par):
        sl = pl.ds(s * RS, RS)
        return ((row_hbm.at[wid, sl], idx_row.at[par]),
                (col_hbm.at[wid, sl], idx_col.at[par]))

    def _stream_refs(s, par):
        out = []
        for j in range(RS):
            src = tbuf.at[par, pl.ds(j * CW, CW)]
            out.append((src, acc_rec.at[idx_row.at[par, j]]))
            out.append((src, acc_sent.at[idx_col.at[par, j]]))
            out.append((ones_v.at[0], acc_cnt.at[idx_row.at[par, j]]))
            out.append((ones_v.at[1], acc_cnt.at[idx_col.at[par, j]]))
        return out

    pltpu.async_copy(*_slab_refs(0), dma_sem)
    for srcdst in _idx_refs(0, 0):
        pltpu.async_copy(*srcdst, dma_sem)

    def slab_step(s, carry):
        par = lax.rem(s, 2)
        pltpu.make_async_copy(*_slab_refs(s), dma_sem).wait()
        for srcdst in _idx_refs(s, par):
            pltpu.make_async_copy(*srcdst, dma_sem).wait()

        def xpose(e, carry):
            vec = plsc.load_gather(xbuf, [lane, jnp.full((16,), e, jnp.int32)])
            plsc.store_scatter(tbuf.at[par], [jnp.full((16,), e, jnp.int32), lane], vec)
            return carry

        lax.fori_loop(0, SB, xpose, 0, unroll=4)

        @pl.when(s > 0)
        def _():
            for src, dst in _stream_refs(s - 1, 1 - par):
                pltpu.make_async_copy(src, dst, str_sem).wait()

        @pl.when(s < SL - 1)
        def _():
            pltpu.async_copy(*_slab_refs(s + 1), dma_sem)
            for srcdst in _idx_refs(s + 1, 1 - par):
                pltpu.async_copy(*srcdst, dma_sem)
        for src, dst in _stream_refs(s, par):
            pltpu.async_copy(src, dst, str_sem, add=True)
        return carry

    lax.fori_loop(0, SL, slab_step, 0)
    for src, dst in _stream_refs(SL - 1, lax.rem(SL - 1, 2)):
        pltpu.make_async_copy(src, dst, str_sem).wait()
    plsc.subcore_barrier()

    pltpu.sync_copy(acc_rec.at[pl.ds(r0, ROWS_T)], rec_out.at[cid, pl.ds(r0, ROWS_T)])
    pltpu.sync_copy(acc_sent.at[pl.ds(r0, ROWS_T)], sent_out.at[cid, pl.ds(r0, ROWS_T)])
    pltpu.sync_copy(acc_cnt.at[pl.ds(r0, ROWS_T)], cnt_out.at[cid, pl.ds(r0, ROWS_T)])


def _make_sc_scatter():
    part = jax.ShapeDtypeStruct((NC, NP, D_EDGE), jnp.float32)
    cpart = jax.ShapeDtypeStruct((NC, NP, 8), jnp.float32)
    return pl.kernel(
        _sc_body,
        out_type=(part, part, cpart),
        mesh=plsc.VectorSubcoreMesh(core_axis_name="c", subcore_axis_name="s",
                                    num_cores=NC, num_subcores=NS),
        compiler_params=pltpu.CompilerParams(use_tc_tiling_on_sc=False,
                                             needs_layout_passes=False),
        scratch_types=[
            pltpu.VMEM((2, RS, CW), jnp.int32),
            pltpu.VMEM((2, RS, CW), jnp.int32),
            pltpu.VMEM((D_EDGE, XW), jnp.float32),
            pltpu.VMEM((2, SB, D_EDGE), jnp.float32),
            pltpu.VMEM((2, CW, 8), jnp.float32),
            pltpu.VMEM_SHARED((NP, D_EDGE), jnp.float32),
            pltpu.VMEM_SHARED((NP, D_EDGE), jnp.float32),
            pltpu.VMEM_SHARED((NP, 8), jnp.float32),
            pltpu.SemaphoreType.DMA,
            pltpu.SemaphoreType.DMA,
        ],
    )


def _tc_body(x_ref, rec_ref, sent_ref, cnt_ref,
             w1x_ref, w1r_ref, w1s_ref, b1_ref, w2_ref, b2_ref, o_ref):
    rec = (rec_ref[0] + rec_ref[1])[:N_NODES]
    sent = (sent_ref[0] + sent_ref[1])[:N_NODES]
    cnt = cnt_ref[0] + cnt_ref[1]
    cr = jnp.maximum(cnt[:N_NODES, 0:1], 1.0)
    cs = jnp.maximum(cnt[:N_NODES, 4:5], 1.0)
    rec = rec / cr
    sent = sent / cs
    h = jnp.dot(x_ref[...], w1x_ref[...], preferred_element_type=jnp.float32)
    h = h + jnp.dot(rec, w1r_ref[...], preferred_element_type=jnp.float32)
    h = h + jnp.dot(sent, w1s_ref[...], preferred_element_type=jnp.float32)
    h = h + b1_ref[...]
    h = jnp.where(h >= 0, h, 0.01 * h)
    o_ref[...] = jnp.dot(h, w2_ref[...], preferred_element_type=jnp.float32) + b2_ref[...]


def kernel(x, edge_index, edge_attr, W1, b1, W2, b2):
    row = edge_index[0].astype(jnp.int32).reshape(NW, KC, CW)
    col = edge_index[1].astype(jnp.int32).reshape(NW, KC, CW)
    z16 = jnp.zeros((NP, D_EDGE), jnp.float32)
    z8 = jnp.zeros((NP, 8), jnp.float32)
    lane = jnp.arange(8)
    on8 = jnp.stack([jnp.where(lane < 4, 1.0, 0.0),
                     jnp.where(lane >= 4, 1.0, 0.0)]).astype(jnp.float32)
    on8 = jnp.broadcast_to(on8[:, None, :], (2, CW, 8))

    rec_p, sent_p, cnt_p = _make_sc_scatter()(row, col, edge_attr.T, z16, z8, on8)

    out = pl.pallas_call(
        _tc_body,
        out_shape=jax.ShapeDtypeStruct((N_NODES, D_NODE), jnp.float32),
    )(x, rec_p, sent_p, cnt_p,
      W1[:D_NODE], W1[D_NODE:D_NODE + D_EDGE], W1[D_NODE + D_EDGE:],
      b1.reshape(1, D_NODE), W2, b2.reshape(1, D_NODE))
    return out


# speedup vs baseline: 13.4680x; 1.0603x over previous
import jax
import jax.numpy as jnp
from jax import lax
from jax.experimental import pallas as pl
from jax.experimental.pallas import tpu as pltpu
from jax.experimental.pallas import tpu_sc as plsc

N_NODES = 10000
N_EDGES = 320000
D_NODE = 128
D_EDGE = 16

NP = 10240
NC = 2
NS = 16
NW = NC * NS
EW = N_EDGES // NW
CW = 125
KC = EW // CW
SB = 1000
XW = SB + 9
RS = SB // CW
SL = EW // SB
ROWS_T = NP // NS


def _sc_body(edge_hbm, attr_hbm, z16_hbm, z8_hbm, on8_hbm,
             rec_out, sent_out, cnt_out,
             idx_row, idx_col, xbuf, tbuf, ones_v,
             acc_rec, acc_sent, acc_cnt, dma_sem, str_sem):
    cid = lax.axis_index("c")
    sid = lax.axis_index("s")
    wid = cid * NS + sid


    pltpu.sync_copy(on8_hbm, ones_v)

    r0 = sid * ROWS_T
    for acc in (acc_rec, acc_sent):
        pltpu.sync_copy(z16_hbm.at[pl.ds(r0, ROWS_T)], acc.at[pl.ds(r0, ROWS_T)])
    pltpu.sync_copy(z8_hbm.at[pl.ds(r0, ROWS_T)], acc_cnt.at[pl.ds(r0, ROWS_T)])
    plsc.subcore_barrier()

    lane = lax.iota(jnp.int32, 16)

    def _slab_refs(s):
        e0 = wid * EW + s * SB
        return attr_hbm.at[:, pl.ds(e0, SB)], xbuf.at[:, pl.ds(0, SB)]

    def _idx_refs(s, par):
        sl = pl.ds(s * RS, RS)
        return ((edge_hbm.at[0, wid, sl], idx_row.at[par]),
                (edge_hbm.at[1, wid, sl], idx_col.at[par]))

    def _stream_refs(s, par):
        out = []
        for j in range(RS):
            src = tbuf.at[par, pl.ds(j * CW, CW)]
            out.append((src, acc_rec.at[idx_row.at[par, j]]))
            out.append((src, acc_sent.at[idx_col.at[par, j]]))
            out.append((ones_v.at[0], acc_cnt.at[idx_row.at[par, j]]))
            out.append((ones_v.at[1], acc_cnt.at[idx_col.at[par, j]]))
        return out

    pltpu.async_copy(*_slab_refs(0), dma_sem)
    for srcdst in _idx_refs(0, 0):
        pltpu.async_copy(*srcdst, dma_sem)

    def slab_step(s, carry):
        par = lax.rem(s, 2)
        pltpu.make_async_copy(*_slab_refs(s), dma_sem).wait()
        for srcdst in _idx_refs(s, par):
            pltpu.make_async_copy(*srcdst, dma_sem).wait()

        def xpose(e, carry):
            vec = plsc.load_gather(xbuf, [lane, jnp.full((16,), e, jnp.int32)])
            plsc.store_scatter(tbuf.at[par], [jnp.full((16,), e, jnp.int32), lane], vec)
            return carry

        lax.fori_loop(0, SB, xpose, 0, unroll=4)

        @pl.when(s > 0)
        def _():
            for src, dst in _stream_refs(s - 1, 1 - par):
                pltpu.make_async_copy(src, dst, str_sem).wait()

        @pl.when(s < SL - 1)
        def _():
            pltpu.async_copy(*_slab_refs(s + 1), dma_sem)
            for srcdst in _idx_refs(s + 1, 1 - par):
                pltpu.async_copy(*srcdst, dma_sem)
        for src, dst in _stream_refs(s, par):
            pltpu.async_copy(src, dst, str_sem, add=True)
        return carry

    lax.fori_loop(0, SL, slab_step, 0)
    for src, dst in _stream_refs(SL - 1, lax.rem(SL - 1, 2)):
        pltpu.make_async_copy(src, dst, str_sem).wait()
    plsc.subcore_barrier()

    pltpu.sync_copy(acc_rec.at[pl.ds(r0, ROWS_T)], rec_out.at[cid, pl.ds(r0, ROWS_T)])
    pltpu.sync_copy(acc_sent.at[pl.ds(r0, ROWS_T)], sent_out.at[cid, pl.ds(r0, ROWS_T)])
    pltpu.sync_copy(acc_cnt.at[pl.ds(r0, ROWS_T)], cnt_out.at[cid, pl.ds(r0, ROWS_T)])


def _make_sc_scatter():
    part = jax.ShapeDtypeStruct((NC, NP, D_EDGE), jnp.float32)
    cpart = jax.ShapeDtypeStruct((NC, NP, 8), jnp.float32)
    return pl.kernel(
        _sc_body,
        out_type=(part, part, cpart),
        mesh=plsc.VectorSubcoreMesh(core_axis_name="c", subcore_axis_name="s",
                                    num_cores=NC, num_subcores=NS),
        compiler_params=pltpu.CompilerParams(use_tc_tiling_on_sc=False,
                                             needs_layout_passes=False),
        scratch_types=[
            pltpu.VMEM((2, RS, CW), jnp.int32),
            pltpu.VMEM((2, RS, CW), jnp.int32),
            pltpu.VMEM((D_EDGE, XW), jnp.float32),
            pltpu.VMEM((2, SB, D_EDGE), jnp.float32),
            pltpu.VMEM((2, CW, 8), jnp.float32),
            pltpu.VMEM_SHARED((NP, D_EDGE), jnp.float32),
            pltpu.VMEM_SHARED((NP, D_EDGE), jnp.float32),
            pltpu.VMEM_SHARED((NP, 8), jnp.float32),
            pltpu.SemaphoreType.DMA,
            pltpu.SemaphoreType.DMA,
        ],
    )


def _hx_body(x_ref, w1x_ref, b1_ref, o_ref):
    o_ref[...] = jnp.dot(x_ref[...], w1x_ref[...],
                         preferred_element_type=jnp.float32) + b1_ref[...]


def _tc_body(hx_ref, rec_ref, sent_ref, cnt_ref,
             w1r_ref, w1s_ref, w2_ref, b2_ref, o_ref):
    rec = (rec_ref[0] + rec_ref[1])[:N_NODES]
    sent = (sent_ref[0] + sent_ref[1])[:N_NODES]
    cnt = cnt_ref[0] + cnt_ref[1]
    cr = jnp.maximum(cnt[:N_NODES, 0:1], 1.0)
    cs = jnp.maximum(cnt[:N_NODES, 4:5], 1.0)
    rec = rec / cr
    sent = sent / cs
    h = hx_ref[...]
    h = h + jnp.dot(rec, w1r_ref[...], preferred_element_type=jnp.float32)
    h = h + jnp.dot(sent, w1s_ref[...], preferred_element_type=jnp.float32)
    h = jnp.where(h >= 0, h, 0.01 * h)
    o_ref[...] = jnp.dot(h, w2_ref[...], preferred_element_type=jnp.float32) + b2_ref[...]


def kernel(x, edge_index, edge_attr, W1, b1, W2, b2):
    edge3 = edge_index.astype(jnp.int32).reshape(2, NW, KC, CW)
    z16 = jnp.zeros((NP, D_EDGE), jnp.float32)
    z8 = jnp.zeros((NP, 8), jnp.float32)
    lane = jnp.arange(8)
    on8 = jnp.stack([jnp.where(lane < 4, 1.0, 0.0),
                     jnp.where(lane >= 4, 1.0, 0.0)]).astype(jnp.float32)
    on8 = jnp.broadcast_to(on8[:, None, :], (2, CW, 8))

    rec_p, sent_p, cnt_p = _make_sc_scatter()(edge3, edge_attr.T, z16, z8, on8)

    hx = pl.pallas_call(
        _hx_body,
        out_shape=jax.ShapeDtypeStruct((N_NODES, D_NODE), jnp.float32),
    )(x, W1[:D_NODE], b1.reshape(1, D_NODE))

    out = pl.pallas_call(
        _tc_body,
        out_shape=jax.ShapeDtypeStruct((N_NODES, D_NODE), jnp.float32),
    )(hx, rec_p, sent_p, cnt_p,
      W1[D_NODE:D_NODE + D_EDGE], W1[D_NODE + D_EDGE:],
      W2, b2.reshape(1, D_NODE))
    return out


# speedup vs baseline: 14.0798x; 1.0454x over previous
import jax
import jax.numpy as jnp
from jax import lax
from jax.experimental import pallas as pl
from jax.experimental.pallas import tpu as pltpu
from jax.experimental.pallas import tpu_sc as plsc

N_NODES = 10000
N_EDGES = 320000
D_NODE = 128
D_EDGE = 16

NP = 10240
NC = 2
NS = 16
NW = NC * NS
EW = N_EDGES // NW
CW = 125
KC = EW // CW
SB = 1000
XW = SB + 9
RS = SB // CW
SL = EW // SB
ROWS_T = NP // NS


DW = 20


def _sc_body(edge_hbm, attr_hbm, z20_hbm,
             rec_out, sent_out,
             idx_row, idx_col, xbuf, tbuf,
             acc_rec, acc_sent, dma_sem, str_sem):
    cid = lax.axis_index("c")
    sid = lax.axis_index("s")
    wid = cid * NS + sid


    r0 = sid * ROWS_T
    for acc in (acc_rec, acc_sent):
        pltpu.sync_copy(z20_hbm.at[pl.ds(r0, ROWS_T)], acc.at[pl.ds(r0, ROWS_T)])

    lane = lax.iota(jnp.int32, 16)
    ones16 = jnp.full((16,), 1.0, jnp.float32)

    def fill_ones(i, carry):
        rows = i * 16 + lane
        for par in range(2):
            for c in range(D_EDGE, DW):
                plsc.store_scatter(tbuf.at[par],
                                   [rows, jnp.full((16,), c, jnp.int32)], ones16)
        return carry

    lax.fori_loop(0, SB // 16, fill_ones, 0, unroll=2)
    plsc.subcore_barrier()

    def _slab_refs(s):
        e0 = wid * EW + s * SB
        return attr_hbm.at[:, pl.ds(e0, SB)], xbuf.at[:, pl.ds(0, SB)]

    def _idx_refs(s, par):
        sl = pl.ds(s * RS, RS)
        return ((edge_hbm.at[0, wid, sl], idx_row.at[par]),
                (edge_hbm.at[1, wid, sl], idx_col.at[par]))

    def _stream_refs(s, par):
        out = []
        for j in range(RS):
            src = tbuf.at[par, pl.ds(j * CW, CW)]
            out.append((src, acc_rec.at[idx_row.at[par, j]]))
            out.append((src, acc_sent.at[idx_col.at[par, j]]))
        return out

    pltpu.async_copy(*_slab_refs(0), dma_sem)
    for srcdst in _idx_refs(0, 0):
        pltpu.async_copy(*srcdst, dma_sem)

    def slab_step(s, carry):
        par = lax.rem(s, 2)
        pltpu.make_async_copy(*_slab_refs(s), dma_sem).wait()
        for srcdst in _idx_refs(s, par):
            pltpu.make_async_copy(*srcdst, dma_sem).wait()

        def xpose(e, carry):
            vec = plsc.load_gather(xbuf, [lane, jnp.full((16,), e, jnp.int32)])
            plsc.store_scatter(tbuf.at[par], [jnp.full((16,), e, jnp.int32), lane], vec)
            return carry

        lax.fori_loop(0, SB, xpose, 0, unroll=4)

        @pl.when(s > 0)
        def _():
            for src, dst in _stream_refs(s - 1, 1 - par):
                pltpu.make_async_copy(src, dst, str_sem).wait()

        @pl.when(s < SL - 1)
        def _():
            pltpu.async_copy(*_slab_refs(s + 1), dma_sem)
            for srcdst in _idx_refs(s + 1, 1 - par):
                pltpu.async_copy(*srcdst, dma_sem)
        for src, dst in _stream_refs(s, par):
            pltpu.async_copy(src, dst, str_sem, add=True)
        return carry

    lax.fori_loop(0, SL, slab_step, 0)
    for src, dst in _stream_refs(SL - 1, lax.rem(SL - 1, 2)):
        pltpu.make_async_copy(src, dst, str_sem).wait()
    plsc.subcore_barrier()

    pltpu.sync_copy(acc_rec.at[pl.ds(r0, ROWS_T)], rec_out.at[cid, pl.ds(r0, ROWS_T)])
    pltpu.sync_copy(acc_sent.at[pl.ds(r0, ROWS_T)], sent_out.at[cid, pl.ds(r0, ROWS_T)])


def _make_sc_scatter():
    part = jax.ShapeDtypeStruct((NC, NP, DW), jnp.float32)
    return pl.kernel(
        _sc_body,
        out_type=(part, part),
        mesh=plsc.VectorSubcoreMesh(core_axis_name="c", subcore_axis_name="s",
                                    num_cores=NC, num_subcores=NS),
        compiler_params=pltpu.CompilerParams(use_tc_tiling_on_sc=False,
                                             needs_layout_passes=False),
        scratch_types=[
            pltpu.VMEM((2, RS, CW), jnp.int32),
            pltpu.VMEM((2, RS, CW), jnp.int32),
            pltpu.VMEM((D_EDGE, XW), jnp.float32),
            pltpu.VMEM((2, SB, DW), jnp.float32),
            pltpu.VMEM_SHARED((NP, DW), jnp.float32),
            pltpu.VMEM_SHARED((NP, DW), jnp.float32),
            pltpu.SemaphoreType.DMA,
            pltpu.SemaphoreType.DMA,
        ],
    )


def _hx_body(x_ref, w1x_ref, b1_ref, o_ref):
    o_ref[...] = jnp.dot(x_ref[...], w1x_ref[...],
                         preferred_element_type=jnp.float32) + b1_ref[...]


def _tc_body(hx_ref, rec_ref, sent_ref,
             w1r_ref, w1s_ref, w2_ref, b2_ref, o_ref):
    recw = (rec_ref[0] + rec_ref[1])[:N_NODES]
    sentw = (sent_ref[0] + sent_ref[1])[:N_NODES]
    cr = jnp.maximum(recw[:, D_EDGE:D_EDGE + 1], 1.0)
    cs = jnp.maximum(sentw[:, D_EDGE:D_EDGE + 1], 1.0)
    rec = recw[:, :D_EDGE] / cr
    sent = sentw[:, :D_EDGE] / cs
    h = hx_ref[...]
    h = h + jnp.dot(rec, w1r_ref[...], preferred_element_type=jnp.float32)
    h = h + jnp.dot(sent, w1s_ref[...], preferred_element_type=jnp.float32)
    h = jnp.where(h >= 0, h, 0.01 * h)
    o_ref[...] = jnp.dot(h, w2_ref[...], preferred_element_type=jnp.float32) + b2_ref[...]


def kernel(x, edge_index, edge_attr, W1, b1, W2, b2):
    edge3 = edge_index.astype(jnp.int32).reshape(2, NW, KC, CW)
    z20 = jnp.zeros((NP, DW), jnp.float32)

    rec_p, sent_p = _make_sc_scatter()(edge3, edge_attr.T, z20)

    hx = pl.pallas_call(
        _hx_body,
        out_shape=jax.ShapeDtypeStruct((N_NODES, D_NODE), jnp.float32),
    )(x, W1[:D_NODE], b1.reshape(1, D_NODE))

    out = pl.pallas_call(
        _tc_body,
        out_shape=jax.ShapeDtypeStruct((N_NODES, D_NODE), jnp.float32),
    )(hx, rec_p, sent_p,
      W1[D_NODE:D_NODE + D_EDGE], W1[D_NODE + D_EDGE:],
      W2, b2.reshape(1, D_NODE))
    return out
